# Initial kernel scaffold; baseline (speedup 1.0000x reference)
import jax
import jax.numpy as jnp
from jax.experimental import pallas as pl


def kernel(x, edge_index, edge_attr, Wl, bl, Wr, br, We, att, bias_out, ln_w, ln_b):
    raise NotImplementedError("write your pallas kernel here")


# speedup vs baseline: 19.1176x; 19.1176x over previous
import jax
import jax.numpy as jnp
from jax import lax
from jax.experimental import pallas as pl
from jax.experimental.pallas import tpu as pltpu
from jax.experimental.pallas import tpu_sc as plsc

F32 = jnp.float32

_N, _E, _D, _H, _C, _DE = 10000, 320000, 128, 4, 32, 16
_HC = _H * _C
_NC, _NS = 2, 16
_NW = _NC * _NS
_NP = 10240
_NP4 = _NP // 4
_STRIPE = _NP // _NS
_EPW = _E // _NW
_BE = 40
_NB = _EPW // _BE


def _proj_body(x_ref, wl_ref, bl_ref, wr_ref, br_ref, xl_ref, xr_ref):
    xb = x_ref[...]
    xl_ref[...] = jnp.dot(xb, wl_ref[...], preferred_element_type=F32) + bl_ref[...]
    xr_ref[...] = jnp.dot(xb, wr_ref[...], preferred_element_type=F32) + br_ref[...]


def _proj(x, Wl, bl, Wr, br):
    R = 1000
    return pl.pallas_call(
        _proj_body,
        grid=(_N // R,),
        in_specs=[
            pl.BlockSpec((R, _D), lambda i: (i, 0)),
            pl.BlockSpec((_D, _HC), lambda i: (0, 0)),
            pl.BlockSpec((1, _HC), lambda i: (0, 0)),
            pl.BlockSpec((_D, _HC), lambda i: (0, 0)),
            pl.BlockSpec((1, _HC), lambda i: (0, 0)),
        ],
        out_specs=[
            pl.BlockSpec((R, _HC), lambda i: (i, 0)),
            pl.BlockSpec((R, _HC), lambda i: (i, 0)),
        ],
        out_shape=[
            jax.ShapeDtypeStruct((_N, _HC), F32),
            jax.ShapeDtypeStruct((_N, _HC), F32),
        ],
    )(x, Wl, bl.reshape(1, _HC), Wr, br.reshape(1, _HC))


def _edge_proj_body(ea_ref, we_ref, ee_ref):
    ee_ref[...] = jnp.dot(ea_ref[...], we_ref[...], preferred_element_type=F32)


def _edge_proj(edge_attr, We):
    R = 8000
    return pl.pallas_call(
        _edge_proj_body,
        grid=(_E // R,),
        in_specs=[
            pl.BlockSpec((R, _DE), lambda i: (i, 0)),
            pl.BlockSpec((_DE, _HC), lambda i: (0, 0)),
        ],
        out_specs=pl.BlockSpec((R, _HC), lambda i: (i, 0)),
        out_shape=jax.ShapeDtypeStruct((_E, _HC), F32),
    )(edge_attr, We)


def _sc_body(xl_hbm, xr_hbm, ee_hbm, eaf_hbm, src_hbm, dst_hbm, att_hbm,
             zw_hbm,
             acc_out, ext_out,
             acc_sp, ext_sp,
             idx_s, idx_d, idx_d4, xl_rows, xr_rows, ee_rows, ea_v,
             msg_buf, ext_buf, att_v,
             sem0, sem1):
    cid = lax.axis_index("c")
    sid = lax.axis_index("s")
    wid = cid * _NS + sid
    r0 = sid * _STRIPE
    e0 = sid * (_NP4 // _NS)

    pltpu.sync_copy(zw_hbm, msg_buf)
    pltpu.sync_copy(zw_hbm, ext_buf)
    for j in range(_STRIPE // _BE):
        pltpu.sync_copy(msg_buf, acc_sp.at[pl.ds(r0 + j * _BE, _BE)])
    for j in range(_NP4 // _NS // _BE):
        pltpu.sync_copy(ext_buf, ext_sp.at[pl.ds(e0 + j * _BE, _BE)])
    pltpu.sync_copy(att_hbm, att_v)
    plsc.subcore_barrier()

    ebase = wid * _EPW
    lane = lax.iota(jnp.int32, 16)
    degv = jnp.where(lane == 4, 1.0, 0.0).astype(F32)

    @pl.loop(0, _NB)
    def block_body(g):
        base = ebase + g * _BE
        pltpu.sync_copy(src_hbm.at[pl.ds(base, _BE)], idx_s)
        pltpu.sync_copy(dst_hbm.at[pl.ds(base, _BE)], idx_d)
        cp_l = pltpu.async_copy(xl_hbm.at[idx_s], xl_rows, sem0)
        cp_r = pltpu.async_copy(xr_hbm.at[idx_d], xr_rows, sem1)
        pltpu.sync_copy(ee_hbm.at[pl.ds(base, _BE)], ee_rows)
        pltpu.sync_copy(eaf_hbm.at[pl.ds(base * _DE, _BE * _DE)], ea_v)
        for j in range(_BE // 16 + (1 if _BE % 16 else 0)):
            o = min(16 * j, _BE - 16)
            v = idx_d[pl.ds(o, 16)]
            plsc.store_scatter(idx_d4, [o + lane], lax.shift_right_logical(v, 2))
        cp_l.wait()
        cp_r.wait()

        def edge_body(e, c):
            wd = degv
            for h in range(_H):
                tv = None
                xls = []
                for q in range(2):
                    off = h * _C + q * 16
                    xlv = xl_rows[e, pl.ds(off, 16)]
                    xls.append(xlv)
                    m = xlv + xr_rows[e, pl.ds(off, 16)] + ee_rows[e, pl.ds(off, 16)]
                    m = jnp.maximum(m, 0.2 * m)
                    t = m * att_v[pl.ds(off, 16)]
                    tv = t if tv is None else tv + t
                a = jnp.sum(tv)
                wv = jnp.exp(jnp.full((16,), a, F32))
                wd = wd + jnp.where(lane == h, wv, 0.0)
                for q in range(2):
                    off = h * _C + q * 16
                    msg_buf[e, pl.ds(off, 16)] = wv * xls[q]
            eav = ea_v[pl.ds(e * _DE, _DE)]
            st = jnp.minimum(e, _BE - 16)
            dvec = idx_d[pl.ds(st, 16)]
            d = jnp.sum(jnp.where(lane == (e - st), dvec, 0))
            dm4 = lax.rem(d, 4)
            for cb in range(4):
                gsel = jnp.where(dm4 == cb, 1.0, 0.0).astype(F32)
                ext_buf[e, pl.ds(32 * cb, 16)] = gsel * eav
                ext_buf[e, pl.ds(32 * cb + 16, 16)] = gsel * wd
            return c
        lax.fori_loop(0, _BE, edge_body, 0)

        pltpu.sync_copy(msg_buf, acc_sp.at[idx_d], add=True)
        pltpu.sync_copy(ext_buf, ext_sp.at[idx_d4], add=True)

    plsc.subcore_barrier()

    o0 = cid * _NP + r0
    for j in range(_STRIPE // _BE):
        pltpu.sync_copy(acc_sp.at[pl.ds(r0 + j * _BE, _BE)], msg_buf)
        pltpu.sync_copy(msg_buf, acc_out.at[pl.ds(o0 + j * _BE, _BE)])
    x0 = cid * _NP4 + e0
    for j in range(_NP4 // _NS // _BE):
        pltpu.sync_copy(ext_sp.at[pl.ds(e0 + j * _BE, _BE)], ext_buf)
        pltpu.sync_copy(ext_buf, ext_out.at[pl.ds(x0 + j * _BE, _BE)])


def _sc_edge_pass(xl, xr, ee, ea_flat, src, dst, att_flat):
    zw = jnp.zeros((_BE, _HC), F32)
    mesh = plsc.VectorSubcoreMesh(core_axis_name="c", subcore_axis_name="s")
    f = pl.kernel(
        _sc_body,
        out_type=[
            jax.ShapeDtypeStruct((_NC * _NP, _HC), F32),
            jax.ShapeDtypeStruct((_NC * _NP4, _HC), F32),
        ],
        mesh=mesh,
        compiler_params=pltpu.CompilerParams(needs_layout_passes=False),
        scratch_types=[
            pltpu.VMEM_SHARED((_NP, _HC), F32),
            pltpu.VMEM_SHARED((_NP4, _HC), F32),
            pltpu.VMEM((_BE,), jnp.int32),
            pltpu.VMEM((_BE,), jnp.int32),
            pltpu.VMEM((_BE,), jnp.int32),
            pltpu.VMEM((_BE, _HC), F32),
            pltpu.VMEM((_BE, _HC), F32),
            pltpu.VMEM((_BE, _HC), F32),
            pltpu.VMEM((_BE * _DE,), F32),
            pltpu.VMEM((_BE, _HC), F32),
            pltpu.VMEM((_BE, _HC), F32),
            pltpu.VMEM((_HC,), F32),
            pltpu.SemaphoreType.DMA,
            pltpu.SemaphoreType.DMA,
        ],
    )
    return f(xl, xr, ee, ea_flat, src, dst, att_flat, zw)


def _combine_body(x_ref, xl_ref, xr_ref, acc_ref, ext_ref,
                  we_ref, att_ref, bias_ref, lnw_ref, lnb_ref, out_ref):
    acc_t = acc_ref[0] + acc_ref[1]
    ext_t = ext_ref[0] + ext_ref[1]
    deg_t = ext_t[:, 20:21]
    la = ext_t[:, 0:_DE] / jnp.maximum(deg_t, 1.0)
    el = jnp.dot(la, we_ref[...], preferred_element_type=F32)
    xlb = xl_ref[...]
    m = xlb + xr_ref[...] + el
    m = jnp.maximum(m, 0.2 * m)
    t = m * att_ref[...]
    outs = []
    for h in range(_H):
        lo = _C * h
        a = jnp.sum(t[:, lo:lo + _C], axis=1, keepdims=True)
        w = jnp.exp(a)
        num = acc_t[:, lo:lo + _C] + w * xlb[:, lo:lo + _C]
        den = ext_t[:, _DE + h:_DE + h + 1] + w + 1e-16
        outs.append(num / den)
    o = jnp.concatenate(outs, axis=1) + bias_ref[...]
    y = x_ref[...] + o
    mu = jnp.mean(y, axis=1, keepdims=True)
    var = jnp.mean((y - mu) * (y - mu), axis=1, keepdims=True)
    y = (y - mu) / jnp.sqrt(var + 1e-5) * lnw_ref[...] + lnb_ref[...]
    out_ref[...] = jnp.maximum(y, 0.0)


def _combine(x, xl, xr, acc, ext, We, att_flat, bias_out, ln_w, ln_b):
    R = 1000
    row = lambda i: (i, 0)
    full2 = lambda i: (0, 0)
    part3 = lambda i: (0, i, 0)
    return pl.pallas_call(
        _combine_body,
        grid=(_N // R,),
        in_specs=[
            pl.BlockSpec((R, _D), row),
            pl.BlockSpec((R, _HC), row),
            pl.BlockSpec((R, _HC), row),
            pl.BlockSpec((_NC, R, _HC), part3),
            pl.BlockSpec((_NC, R, 32), part3),
            pl.BlockSpec((_DE, _HC), full2),
            pl.BlockSpec((1, _HC), full2),
            pl.BlockSpec((1, _HC), full2),
            pl.BlockSpec((1, _HC), full2),
            pl.BlockSpec((1, _HC), full2),
        ],
        out_specs=pl.BlockSpec((R, _HC), row),
        out_shape=jax.ShapeDtypeStruct((_N, _HC), F32),
    )(x, xl, xr, acc, ext, We,
      att_flat.reshape(1, _HC), bias_out.reshape(1, _HC),
      ln_w.reshape(1, _HC), ln_b.reshape(1, _HC))


def kernel(x, edge_index, edge_attr, Wl, bl, Wr, br, We, att, bias_out, ln_w, ln_b):
    src = edge_index[0]
    dst = edge_index[1]
    xl, xr = _proj(x, Wl, bl, Wr, br)
    ee = _edge_proj(edge_attr, We)
    att_flat = att.reshape(_HC)
    acc_f, ext_f = _sc_edge_pass(xl, xr, ee, edge_attr.reshape(_E * _DE), src, dst, att_flat)
    acc = acc_f.reshape(_NC, _NP, _HC)
    ext = ext_f.reshape(_NC, _NP, 32)
    return _combine(x, xl, xr, acc, ext, We, att_flat, bias_out, ln_w, ln_b)


# speedup vs baseline: 19.1927x; 1.0039x over previous
import jax
import jax.numpy as jnp
from jax import lax
from jax.experimental import pallas as pl
from jax.experimental.pallas import tpu as pltpu
from jax.experimental.pallas import tpu_sc as plsc

F32 = jnp.float32

_N, _E, _D, _H, _C, _DE = 10000, 320000, 128, 4, 32, 16
_HC = _H * _C
_NC, _NS = 2, 16
_NW = _NC * _NS
_NP = 10240
_NP4 = _NP // 4
_STRIPE = _NP // _NS
_EPW = _E // _NW
_BE = 40
_NB = _EPW // _BE
_SBB = 10


def _proj_body(x_ref, wl_ref, bl_ref, wr_ref, br_ref, xl_ref, xr_ref):
    xb = x_ref[...]
    xl_ref[...] = jnp.dot(xb, wl_ref[...], preferred_element_type=F32) + bl_ref[...]
    xr_ref[...] = jnp.dot(xb, wr_ref[...], preferred_element_type=F32) + br_ref[...]


def _proj(x, Wl, bl, Wr, br):
    R = 1000
    return pl.pallas_call(
        _proj_body,
        grid=(_N // R,),
        in_specs=[
            pl.BlockSpec((R, _D), lambda i: (i, 0)),
            pl.BlockSpec((_D, _HC), lambda i: (0, 0)),
            pl.BlockSpec((1, _HC), lambda i: (0, 0)),
            pl.BlockSpec((_D, _HC), lambda i: (0, 0)),
            pl.BlockSpec((1, _HC), lambda i: (0, 0)),
        ],
        out_specs=[
            pl.BlockSpec((R, _HC), lambda i: (i, 0)),
            pl.BlockSpec((R, _HC), lambda i: (i, 0)),
        ],
        out_shape=[
            jax.ShapeDtypeStruct((_N, _HC), F32),
            jax.ShapeDtypeStruct((_N, _HC), F32),
        ],
    )(x, Wl, bl.reshape(1, _HC), Wr, br.reshape(1, _HC))


def _edge_proj_body(ea_ref, we_ref, ee_ref):
    ee_ref[...] = jnp.dot(ea_ref[...], we_ref[...], preferred_element_type=F32)


def _edge_proj(edge_attr, We):
    R = 8000
    return pl.pallas_call(
        _edge_proj_body,
        grid=(_E // R,),
        in_specs=[
            pl.BlockSpec((R, _DE), lambda i: (i, 0)),
            pl.BlockSpec((_DE, _HC), lambda i: (0, 0)),
        ],
        out_specs=pl.BlockSpec((R, _HC), lambda i: (i, 0)),
        out_shape=jax.ShapeDtypeStruct((_E, _HC), F32),
    )(edge_attr, We)


def _sc_body(xl_hbm, xr_hbm, ee_hbm, eaf_hbm, src_hbm, dst_hbm, att_hbm,
             zw_hbm,
             acc_out, ext_out,
             acc_sp, ext_sp,
             idx_sb, idx_db, idx_s0, idx_s1, idx_d0, idx_d1, idx_d4,
             xl0, xl1, xr0, xr1, ee_rows, ea_v, att_v,
             sxl0, sxl1, sxr0, sxr1):
    cid = lax.axis_index("c")
    sid = lax.axis_index("s")
    wid = cid * _NS + sid
    r0 = sid * _STRIPE
    e0 = sid * (_NP4 // _NS)

    pltpu.sync_copy(zw_hbm, xl0)
    for j in range(_STRIPE // _BE):
        pltpu.sync_copy(xl0, acc_sp.at[pl.ds(r0 + j * _BE, _BE)])
    for j in range(_NP4 // _NS // _BE):
        pltpu.sync_copy(xl0, ext_sp.at[pl.ds(e0 + j * _BE, _BE)])
    pltpu.sync_copy(att_hbm, att_v)
    plsc.subcore_barrier()

    ebase = wid * _EPW
    lane = lax.iota(jnp.int32, 16)
    degv = jnp.where(lane == 4, 1.0, 0.0).astype(F32)

    def prefetch(pb, idx_s_slot, idx_d_slot, xl_slot, xr_slot, sa, sb):
        @pl.when(pb < _NB)
        def _():
            @pl.when(lax.rem(pb, _SBB) == 0)
            def _():
                pltpu.sync_copy(src_hbm.at[pl.ds(ebase + pb * _BE, _SBB * _BE)], idx_sb)
                pltpu.sync_copy(dst_hbm.at[pl.ds(ebase + pb * _BE, _SBB * _BE)], idx_db)
            off0 = lax.rem(pb, _SBB) * _BE
            for o in (0, 16, 24):
                plsc.store_scatter(idx_s_slot, [o + lane], idx_sb[pl.ds(off0 + o, 16)])
                plsc.store_scatter(idx_d_slot, [o + lane], idx_db[pl.ds(off0 + o, 16)])
            pltpu.async_copy(xl_hbm.at[idx_s_slot], xl_slot, sa)
            pltpu.async_copy(xr_hbm.at[idx_d_slot], xr_slot, sb)

    def process(pb, idx_s_slot, idx_d_slot, xl_slot, xr_slot, sa, sb):
        base = ebase + pb * _BE
        pltpu.sync_copy(ee_hbm.at[pl.ds(base, _BE)], ee_rows)
        pltpu.sync_copy(eaf_hbm.at[pl.ds(base * _DE, _BE * _DE)], ea_v)
        for o in (0, 16, 24):
            v = idx_d_slot[pl.ds(o, 16)]
            plsc.store_scatter(idx_d4, [o + lane], lax.shift_right_logical(v, 2))
        pltpu.make_async_copy(xl_hbm.at[idx_s_slot], xl_slot, sa).wait()
        pltpu.make_async_copy(xr_hbm.at[idx_d_slot], xr_slot, sb).wait()

        def edge_body(e, c):
            wd = degv
            for h in range(_H):
                tv = None
                xls = []
                for q in range(2):
                    off = h * _C + q * 16
                    xlv = xl_slot[e, pl.ds(off, 16)]
                    xls.append(xlv)
                    m = xlv + xr_slot[e, pl.ds(off, 16)] + ee_rows[e, pl.ds(off, 16)]
                    m = jnp.maximum(m, 0.2 * m)
                    t = m * att_v[pl.ds(off, 16)]
                    tv = t if tv is None else tv + t
                a = jnp.sum(tv)
                wv = jnp.exp(jnp.full((16,), a, F32))
                wd = wd + jnp.where(lane == h, wv, 0.0)
                for q in range(2):
                    off = h * _C + q * 16
                    xl_slot[e, pl.ds(off, 16)] = wv * xls[q]
            eav = ea_v[pl.ds(e * _DE, _DE)]
            st = jnp.minimum(e, _BE - 16)
            dvec = idx_d_slot[pl.ds(st, 16)]
            d = jnp.sum(jnp.where(lane == (e - st), dvec, 0))
            dm4 = lax.rem(d, 4)
            for cb in range(4):
                gsel = jnp.where(dm4 == cb, 1.0, 0.0).astype(F32)
                ee_rows[e, pl.ds(32 * cb, 16)] = gsel * eav
                ee_rows[e, pl.ds(32 * cb + 16, 16)] = gsel * wd
            return c
        lax.fori_loop(0, _BE, edge_body, 0)

        pltpu.sync_copy(xl_slot, acc_sp.at[idx_d_slot], add=True)
        pltpu.sync_copy(ee_rows, ext_sp.at[idx_d4], add=True)

    prefetch(0, idx_s0, idx_d0, xl0, xr0, sxl0, sxr0)

    @pl.loop(0, _NB, step=2)
    def block_body(g):
        prefetch(g + 1, idx_s1, idx_d1, xl1, xr1, sxl1, sxr1)
        process(g, idx_s0, idx_d0, xl0, xr0, sxl0, sxr0)
        prefetch(g + 2, idx_s0, idx_d0, xl0, xr0, sxl0, sxr0)
        process(g + 1, idx_s1, idx_d1, xl1, xr1, sxl1, sxr1)

    plsc.subcore_barrier()

    o0 = cid * _NP + r0
    for j in range(_STRIPE // _BE):
        pltpu.sync_copy(acc_sp.at[pl.ds(r0 + j * _BE, _BE)], xl0)
        pltpu.sync_copy(xl0, acc_out.at[pl.ds(o0 + j * _BE, _BE)])
    x0 = cid * _NP4 + e0
    for j in range(_NP4 // _NS // _BE):
        pltpu.sync_copy(ext_sp.at[pl.ds(e0 + j * _BE, _BE)], xl0)
        pltpu.sync_copy(xl0, ext_out.at[pl.ds(x0 + j * _BE, _BE)])


def _sc_edge_pass(xl, xr, ee, ea_flat, src, dst, att_flat):
    zw = jnp.zeros((_BE, _HC), F32)
    mesh = plsc.VectorSubcoreMesh(core_axis_name="c", subcore_axis_name="s")
    f = pl.kernel(
        _sc_body,
        out_type=[
            jax.ShapeDtypeStruct((_NC * _NP, _HC), F32),
            jax.ShapeDtypeStruct((_NC * _NP4, _HC), F32),
        ],
        mesh=mesh,
        compiler_params=pltpu.CompilerParams(needs_layout_passes=False),
        scratch_types=[
            pltpu.VMEM_SHARED((_NP, _HC), F32),
            pltpu.VMEM_SHARED((_NP4, _HC), F32),
            pltpu.VMEM((_SBB * _BE,), jnp.int32),
            pltpu.VMEM((_SBB * _BE,), jnp.int32),
            pltpu.VMEM((_BE,), jnp.int32),
            pltpu.VMEM((_BE,), jnp.int32),
            pltpu.VMEM((_BE,), jnp.int32),
            pltpu.VMEM((_BE,), jnp.int32),
            pltpu.VMEM((_BE,), jnp.int32),
            pltpu.VMEM((_BE, _HC), F32),
            pltpu.VMEM((_BE, _HC), F32),
            pltpu.VMEM((_BE, _HC), F32),
            pltpu.VMEM((_BE, _HC), F32),
            pltpu.VMEM((_BE, _HC), F32),
            pltpu.VMEM((_BE * _DE,), F32),
            pltpu.VMEM((_HC,), F32),
            pltpu.SemaphoreType.DMA,
            pltpu.SemaphoreType.DMA,
            pltpu.SemaphoreType.DMA,
            pltpu.SemaphoreType.DMA,
        ],
    )
    return f(xl, xr, ee, ea_flat, src, dst, att_flat, zw)


def _combine_body(x_ref, xl_ref, xr_ref, acc_ref, ext_ref,
                  we_ref, att_ref, bias_ref, lnw_ref, lnb_ref, out_ref):
    acc_t = acc_ref[0] + acc_ref[1]
    ext_t = ext_ref[0] + ext_ref[1]
    deg_t = ext_t[:, 20:21]
    la = ext_t[:, 0:_DE] / jnp.maximum(deg_t, 1.0)
    el = jnp.dot(la, we_ref[...], preferred_element_type=F32)
    xlb = xl_ref[...]
    m = xlb + xr_ref[...] + el
    m = jnp.maximum(m, 0.2 * m)
    t = m * att_ref[...]
    outs = []
    for h in range(_H):
        lo = _C * h
        a = jnp.sum(t[:, lo:lo + _C], axis=1, keepdims=True)
        w = jnp.exp(a)
        num = acc_t[:, lo:lo + _C] + w * xlb[:, lo:lo + _C]
        den = ext_t[:, _DE + h:_DE + h + 1] + w + 1e-16
        outs.append(num / den)
    o = jnp.concatenate(outs, axis=1) + bias_ref[...]
    y = x_ref[...] + o
    mu = jnp.mean(y, axis=1, keepdims=True)
    var = jnp.mean((y - mu) * (y - mu), axis=1, keepdims=True)
    y = (y - mu) / jnp.sqrt(var + 1e-5) * lnw_ref[...] + lnb_ref[...]
    out_ref[...] = jnp.maximum(y, 0.0)


def _combine(x, xl, xr, acc, ext, We, att_flat, bias_out, ln_w, ln_b):
    R = 1000
    row = lambda i: (i, 0)
    full2 = lambda i: (0, 0)
    part3 = lambda i: (0, i, 0)
    return pl.pallas_call(
        _combine_body,
        grid=(_N // R,),
        in_specs=[
            pl.BlockSpec((R, _D), row),
            pl.BlockSpec((R, _HC), row),
            pl.BlockSpec((R, _HC), row),
            pl.BlockSpec((_NC, R, _HC), part3),
            pl.BlockSpec((_NC, R, 32), part3),
            pl.BlockSpec((_DE, _HC), full2),
            pl.BlockSpec((1, _HC), full2),
            pl.BlockSpec((1, _HC), full2),
            pl.BlockSpec((1, _HC), full2),
            pl.BlockSpec((1, _HC), full2),
        ],
        out_specs=pl.BlockSpec((R, _HC), row),
        out_shape=jax.ShapeDtypeStruct((_N, _HC), F32),
    )(x, xl, xr, acc, ext, We,
      att_flat.reshape(1, _HC), bias_out.reshape(1, _HC),
      ln_w.reshape(1, _HC), ln_b.reshape(1, _HC))


def kernel(x, edge_index, edge_attr, Wl, bl, Wr, br, We, att, bias_out, ln_w, ln_b):
    src = edge_index[0]
    dst = edge_index[1]
    xl, xr = _proj(x, Wl, bl, Wr, br)
    ee = _edge_proj(edge_attr, We)
    att_flat = att.reshape(_HC)
    acc_f, ext_f = _sc_edge_pass(xl, xr, ee, edge_attr.reshape(_E * _DE), src, dst, att_flat)
    acc = acc_f.reshape(_NC, _NP, _HC)
    ext = ext_f.reshape(_NC, _NP, 32)
    return _combine(x, xl, xr, acc, ext, We, att_flat, bias_out, ln_w, ln_b)


# speedup vs baseline: 19.3140x; 1.0063x over previous
import jax
import jax.numpy as jnp
from jax import lax
from jax.experimental import pallas as pl
from jax.experimental.pallas import tpu as pltpu
from jax.experimental.pallas import tpu_sc as plsc

F32 = jnp.float32

_N, _E, _D, _H, _C, _DE = 10000, 320000, 128, 4, 32, 16
_HC = _H * _C
_NC, _NS = 2, 16
_NW = _NC * _NS
_NP = 10240
_NP4 = _NP // 4
_STRIPE = _NP // _NS
_EPW = _E // _NW
_BE = 40
_NB = _EPW // _BE
_SBB = 10


def _proj_body(x_ref, wl_ref, bl_ref, wr_ref, br_ref, xl_ref, xr_ref):
    xb = x_ref[...]
    xl_ref[...] = jnp.dot(xb, wl_ref[...], preferred_element_type=F32) + bl_ref[...]
    xr_ref[...] = jnp.dot(xb, wr_ref[...], preferred_element_type=F32) + br_ref[...]


def _proj(x, Wl, bl, Wr, br):
    R = 1000
    return pl.pallas_call(
        _proj_body,
        grid=(_N // R,),
        in_specs=[
            pl.BlockSpec((R, _D), lambda i: (i, 0)),
            pl.BlockSpec((_D, _HC), lambda i: (0, 0)),
            pl.BlockSpec((1, _HC), lambda i: (0, 0)),
            pl.BlockSpec((_D, _HC), lambda i: (0, 0)),
            pl.BlockSpec((1, _HC), lambda i: (0, 0)),
        ],
        out_specs=[
            pl.BlockSpec((R, _HC), lambda i: (i, 0)),
            pl.BlockSpec((R, _HC), lambda i: (i, 0)),
        ],
        out_shape=[
            jax.ShapeDtypeStruct((_N, _HC), F32),
            jax.ShapeDtypeStruct((_N, _HC), F32),
        ],
    )(x, Wl, bl.reshape(1, _HC), Wr, br.reshape(1, _HC))


def _edge_proj_body(ea_ref, we_ref, ee_ref):
    ee_ref[...] = jnp.dot(ea_ref[...], we_ref[...], preferred_element_type=F32)


def _edge_proj(edge_attr, We):
    R = 8000
    return pl.pallas_call(
        _edge_proj_body,
        grid=(_E // R,),
        in_specs=[
            pl.BlockSpec((R, _DE), lambda i: (i, 0)),
            pl.BlockSpec((_DE, _HC), lambda i: (0, 0)),
        ],
        out_specs=pl.BlockSpec((R, _HC), lambda i: (i, 0)),
        out_shape=jax.ShapeDtypeStruct((_E, _HC), F32),
    )(edge_attr, We)


def _sc_body(xl_hbm, xr_hbm, ee_hbm, eaf_hbm, src_hbm, dst_hbm, att_hbm,
             zw_hbm,
             acc_out, ext_out,
             acc_sp, ext_sp,
             idx_sb, idx_db, idx_s0, idx_s1, idx_d0, idx_d1, idx_d4,
             xl0, xl1, xr0, xr1, ee_rows, ea_v, att_v,
             sxl0, sxl1, sxr0, sxr1):
    cid = lax.axis_index("c")
    sid = lax.axis_index("s")
    wid = cid * _NS + sid
    r0 = sid * _STRIPE
    e0 = sid * (_NP4 // _NS)

    pltpu.sync_copy(zw_hbm, xl0)
    for j in range(_STRIPE // _BE):
        pltpu.sync_copy(xl0, acc_sp.at[pl.ds(r0 + j * _BE, _BE)])
    for j in range(_NP4 // _NS // _BE):
        pltpu.sync_copy(xl0, ext_sp.at[pl.ds(e0 + j * _BE, _BE)])
    pltpu.sync_copy(att_hbm, att_v)
    plsc.subcore_barrier()

    ebase = wid * _EPW
    lane = lax.iota(jnp.int32, 16)
    degv = jnp.where(lane == 4, 1.0, 0.0).astype(F32)

    def prefetch(pb, idx_s_slot, idx_d_slot, xl_slot, xr_slot, sa, sb):
        @pl.when(pb < _NB)
        def _():
            @pl.when(lax.rem(pb, _SBB) == 0)
            def _():
                pltpu.sync_copy(src_hbm.at[pl.ds(ebase + pb * _BE, _SBB * _BE)], idx_sb)
                pltpu.sync_copy(dst_hbm.at[pl.ds(ebase + pb * _BE, _SBB * _BE)], idx_db)
            off0 = lax.rem(pb, _SBB) * _BE
            for o in (0, 16, 24):
                plsc.store_scatter(idx_s_slot, [o + lane], idx_sb[pl.ds(off0 + o, 16)])
                plsc.store_scatter(idx_d_slot, [o + lane], idx_db[pl.ds(off0 + o, 16)])
            pltpu.async_copy(xl_hbm.at[idx_s_slot], xl_slot, sa)
            pltpu.async_copy(xr_hbm.at[idx_d_slot], xr_slot, sb)

    def process(pb, idx_s_slot, idx_d_slot, xl_slot, xr_slot, sa, sb):
        base = ebase + pb * _BE
        pltpu.sync_copy(ee_hbm.at[pl.ds(base, _BE)], ee_rows)
        pltpu.sync_copy(eaf_hbm.at[pl.ds(base * _DE, _BE * _DE)], ea_v)
        for o in (0, 16, 24):
            v = idx_d_slot[pl.ds(o, 16)]
            plsc.store_scatter(idx_d4, [o + lane], lax.shift_right_logical(v, 2))
        pltpu.make_async_copy(xl_hbm.at[idx_s_slot], xl_slot, sa).wait()
        pltpu.make_async_copy(xr_hbm.at[idx_d_slot], xr_slot, sb).wait()

        def edge_body(e, c):
            wd = degv
            for h in range(_H):
                tv = None
                xls = []
                for q in range(2):
                    off = h * _C + q * 16
                    xlv = xl_slot[e, pl.ds(off, 16)]
                    xls.append(xlv)
                    m = xlv + xr_slot[e, pl.ds(off, 16)] + ee_rows[e, pl.ds(off, 16)]
                    m = jnp.maximum(m, 0.2 * m)
                    t = m * att_v[pl.ds(off, 16)]
                    tv = t if tv is None else tv + t
                a = jnp.sum(tv)
                wv = jnp.exp(jnp.full((16,), a, F32))
                wd = wd + jnp.where(lane == h, wv, 0.0)
                for q in range(2):
                    off = h * _C + q * 16
                    xl_slot[e, pl.ds(off, 16)] = wv * xls[q]
            eav = ea_v[pl.ds(e * _DE, _DE)]
            st = jnp.minimum(e, _BE - 16)
            dvec = idx_d_slot[pl.ds(st, 16)]
            d = jnp.sum(jnp.where(lane == (e - st), dvec, 0))
            dm4 = lax.rem(d, 4)
            for cb in range(4):
                gsel = jnp.where(dm4 == cb, 1.0, 0.0).astype(F32)
                ee_rows[e, pl.ds(32 * cb, 16)] = gsel * eav
                ee_rows[e, pl.ds(32 * cb + 16, 16)] = gsel * wd
            return c
        lax.fori_loop(0, _BE, edge_body, 0, unroll=4)

        pltpu.sync_copy(xl_slot, acc_sp.at[idx_d_slot], add=True)
        pltpu.sync_copy(ee_rows, ext_sp.at[idx_d4], add=True)

    prefetch(0, idx_s0, idx_d0, xl0, xr0, sxl0, sxr0)

    @pl.loop(0, _NB, step=2)
    def block_body(g):
        prefetch(g + 1, idx_s1, idx_d1, xl1, xr1, sxl1, sxr1)
        process(g, idx_s0, idx_d0, xl0, xr0, sxl0, sxr0)
        prefetch(g + 2, idx_s0, idx_d0, xl0, xr0, sxl0, sxr0)
        process(g + 1, idx_s1, idx_d1, xl1, xr1, sxl1, sxr1)

    plsc.subcore_barrier()

    o0 = cid * _NP + r0
    for j in range(_STRIPE // _BE):
        pltpu.sync_copy(acc_sp.at[pl.ds(r0 + j * _BE, _BE)], xl0)
        pltpu.sync_copy(xl0, acc_out.at[pl.ds(o0 + j * _BE, _BE)])
    x0 = cid * _NP4 + e0
    for j in range(_NP4 // _NS // _BE):
        pltpu.sync_copy(ext_sp.at[pl.ds(e0 + j * _BE, _BE)], xl0)
        pltpu.sync_copy(xl0, ext_out.at[pl.ds(x0 + j * _BE, _BE)])


def _sc_edge_pass(xl, xr, ee, ea_flat, src, dst, att_flat):
    zw = jnp.zeros((_BE, _HC), F32)
    mesh = plsc.VectorSubcoreMesh(core_axis_name="c", subcore_axis_name="s")
    f = pl.kernel(
        _sc_body,
        out_type=[
            jax.ShapeDtypeStruct((_NC * _NP, _HC), F32),
            jax.ShapeDtypeStruct((_NC * _NP4, _HC), F32),
        ],
        mesh=mesh,
        compiler_params=pltpu.CompilerParams(needs_layout_passes=False),
        scratch_types=[
            pltpu.VMEM_SHARED((_NP, _HC), F32),
            pltpu.VMEM_SHARED((_NP4, _HC), F32),
            pltpu.VMEM((_SBB * _BE,), jnp.int32),
            pltpu.VMEM((_SBB * _BE,), jnp.int32),
            pltpu.VMEM((_BE,), jnp.int32),
            pltpu.VMEM((_BE,), jnp.int32),
            pltpu.VMEM((_BE,), jnp.int32),
            pltpu.VMEM((_BE,), jnp.int32),
            pltpu.VMEM((_BE,), jnp.int32),
            pltpu.VMEM((_BE, _HC), F32),
            pltpu.VMEM((_BE, _HC), F32),
            pltpu.VMEM((_BE, _HC), F32),
            pltpu.VMEM((_BE, _HC), F32),
            pltpu.VMEM((_BE, _HC), F32),
            pltpu.VMEM((_BE * _DE,), F32),
            pltpu.VMEM((_HC,), F32),
            pltpu.SemaphoreType.DMA,
            pltpu.SemaphoreType.DMA,
            pltpu.SemaphoreType.DMA,
            pltpu.SemaphoreType.DMA,
        ],
    )
    return f(xl, xr, ee, ea_flat, src, dst, att_flat, zw)


def _combine_body(x_ref, xl_ref, xr_ref, acc_ref, ext_ref,
                  we_ref, att_ref, bias_ref, lnw_ref, lnb_ref, out_ref):
    acc_t = acc_ref[0] + acc_ref[1]
    ext_t = ext_ref[0] + ext_ref[1]
    deg_t = ext_t[:, 20:21]
    la = ext_t[:, 0:_DE] / jnp.maximum(deg_t, 1.0)
    el = jnp.dot(la, we_ref[...], preferred_element_type=F32)
    xlb = xl_ref[...]
    m = xlb + xr_ref[...] + el
    m = jnp.maximum(m, 0.2 * m)
    t = m * att_ref[...]
    outs = []
    for h in range(_H):
        lo = _C * h
        a = jnp.sum(t[:, lo:lo + _C], axis=1, keepdims=True)
        w = jnp.exp(a)
        num = acc_t[:, lo:lo + _C] + w * xlb[:, lo:lo + _C]
        den = ext_t[:, _DE + h:_DE + h + 1] + w + 1e-16
        outs.append(num / den)
    o = jnp.concatenate(outs, axis=1) + bias_ref[...]
    y = x_ref[...] + o
    mu = jnp.mean(y, axis=1, keepdims=True)
    var = jnp.mean((y - mu) * (y - mu), axis=1, keepdims=True)
    y = (y - mu) / jnp.sqrt(var + 1e-5) * lnw_ref[...] + lnb_ref[...]
    out_ref[...] = jnp.maximum(y, 0.0)


def _combine(x, xl, xr, acc, ext, We, att_flat, bias_out, ln_w, ln_b):
    R = 1000
    row = lambda i: (i, 0)
    full2 = lambda i: (0, 0)
    part3 = lambda i: (0, i, 0)
    return pl.pallas_call(
        _combine_body,
        grid=(_N // R,),
        in_specs=[
            pl.BlockSpec((R, _D), row),
            pl.BlockSpec((R, _HC), row),
            pl.BlockSpec((R, _HC), row),
            pl.BlockSpec((_NC, R, _HC), part3),
            pl.BlockSpec((_NC, R, 32), part3),
            pl.BlockSpec((_DE, _HC), full2),
            pl.BlockSpec((1, _HC), full2),
            pl.BlockSpec((1, _HC), full2),
            pl.BlockSpec((1, _HC), full2),
            pl.BlockSpec((1, _HC), full2),
        ],
        out_specs=pl.BlockSpec((R, _HC), row),
        out_shape=jax.ShapeDtypeStruct((_N, _HC), F32),
    )(x, xl, xr, acc, ext, We,
      att_flat.reshape(1, _HC), bias_out.reshape(1, _HC),
      ln_w.reshape(1, _HC), ln_b.reshape(1, _HC))


def kernel(x, edge_index, edge_attr, Wl, bl, Wr, br, We, att, bias_out, ln_w, ln_b):
    src = edge_index[0]
    dst = edge_index[1]
    xl, xr = _proj(x, Wl, bl, Wr, br)
    ee = _edge_proj(edge_attr, We)
    att_flat = att.reshape(_HC)
    acc_f, ext_f = _sc_edge_pass(xl, xr, ee, edge_attr.reshape(_E * _DE), src, dst, att_flat)
    acc = acc_f.reshape(_NC, _NP, _HC)
    ext = ext_f.reshape(_NC, _NP, 32)
    return _combine(x, xl, xr, acc, ext, We, att_flat, bias_out, ln_w, ln_b)
